# baseline (speedup 1.0000x reference)
import jax
import jax.numpy as jnp
from jax import lax
from jax.experimental import pallas as pl
from jax.experimental.pallas import tpu as pltpu
from jax.experimental.pallas import tpu_sc as plsc

NC = 2
NS = 16
NW = NC * NS
HT = 25
BT = 32
TILES_PER_TEC = HT * BT // NW


def _body(xt_hbm, tp_hbm, out_hbm, xidx_v, pidx_v, poff_v, rows0, rows1,
          ot0, ot1, gsem0, gsem1, osem0, osem1):
    wid = lax.axis_index("s") * NC + lax.axis_index("c")
    base_t = wid * TILES_PER_TEC

    lane = lax.iota(jnp.int32, 16)
    bidx = [lane + 16 * g for g in range(8)]

    rows = (rows0, rows1)
    ots = (ot0, ot1)
    gsems = (gsem0, gsem1)
    osems = (osem0, osem1)

    @pl.loop(0, TILES_PER_TEC)
    def _tile(k):
        t = base_t + k
        ht = lax.shift_right_logical(t, 5)
        bt = lax.bitwise_and(t, BT - 1)

        pltpu.sync_copy(xt_hbm.at[pl.ds(ht * 8, 8), pl.ds(bt * 128, 128)],
                        xidx_v)
        for r8 in range(8):
            for g8 in range(8):
                v = xidx_v[r8, pl.ds(g8 * 16, 16)]
                pidx_v[r8, pl.ds(g8 * 16, 16)] = lax.shift_right_logical(v, 1)
                poff_v[r8, pl.ds(g8 * 16, 16)] = lax.shift_left(
                    lax.bitwise_and(v, 1), 6)

        gdesc = [None] * 8
        wdesc = [None] * 8
        gdesc[0] = pltpu.async_copy(tp_hbm.at[pidx_v.at[0]], rows[0], gsem0)
        for r in range(8):
            buf = r % 2
            if r < 7:
                gdesc[r + 1] = pltpu.async_copy(
                    tp_hbm.at[pidx_v.at[r + 1]], rows[(r + 1) % 2],
                    gsems[(r + 1) % 2])
            gdesc[r].wait()
            if r >= 2:
                wdesc[r - 2].wait()
            for g in range(8):
                pv = poff_v[r, pl.ds(g * 16, 16)]

                @pl.loop(0, 8)
                def _dt(dt, pv=pv, g=g, buf=buf):
                    dbase = dt * 8
                    for d in range(8):
                        col = pv + (dbase + d)
                        ots[buf][dbase + d, pl.ds(g * 16, 16)] = (
                            plsc.load_gather(rows[buf], [bidx[g], col]))

            wdesc[r] = pltpu.async_copy(
                ots[buf], out_hbm.at[ht * 8 + r, :, pl.ds(bt * 128, 128)],
                osems[buf])
        wdesc[6].wait()
        wdesc[7].wait()


def kernel(x, table):
    mesh = plsc.VectorSubcoreMesh(core_axis_name="c", subcore_axis_name="s")
    out = pl.kernel(
        _body,
        out_type=jax.ShapeDtypeStruct((200, 64, 4096), jnp.float32),
        mesh=mesh,
        scratch_types=[
            pltpu.VMEM((8, 128), jnp.int32),
            pltpu.VMEM((8, 128), jnp.int32),
            pltpu.VMEM((8, 128), jnp.int32),
            pltpu.VMEM((128, 128), jnp.float32),
            pltpu.VMEM((128, 128), jnp.float32),
            pltpu.VMEM((64, 128), jnp.float32),
            pltpu.VMEM((64, 128), jnp.float32),
            pltpu.SemaphoreType.DMA,
            pltpu.SemaphoreType.DMA,
            pltpu.SemaphoreType.DMA,
            pltpu.SemaphoreType.DMA,
        ],
        compiler_params=pltpu.CompilerParams(
            use_tc_tiling_on_sc=True, needs_layout_passes=False),
    )(x.T, table.reshape(500000, 128))
    return out.transpose(2, 0, 1)
